# Initial kernel scaffold; baseline (speedup 1.0000x reference)
import jax
import jax.numpy as jnp
from jax.experimental import pallas as pl


def kernel(data_x, data_edge_index, data_edge_attr, data_batch, frag_x, frag_edge_index, frag_edge_attr, frag_batch, junction_x, junction_edge_index, junction_edge_attr, junction_batch, motif_nodes, global_feats, params):
    raise NotImplementedError("write your pallas kernel here")


# speedup vs baseline: 1.0953x; 1.0953x over previous
import functools

import jax
import jax.numpy as jnp
from jax import lax
from jax.experimental import pallas as pl
from jax.experimental.pallas import tpu as pltpu

N_MOLS = 256
N_JCT = 2048
N_HEADS = 2
N_ATOM_LAYERS = 2
N_MOL_LAYERS = 2


def _gru(x, h, q):
    r = jax.nn.sigmoid(x @ q['Wr'] + h @ q['Ur'] + q['br'])
    z = jax.nn.sigmoid(x @ q['Wz'] + h @ q['Uz'] + q['bz'])
    n = jnp.tanh(x @ q['Wn'] + r * (h @ q['Un']) + q['bn'])
    return (1.0 - z) * n + z * h


def _seg_softmax(s, seg, num):
    m = jax.ops.segment_max(s, seg, num)
    m = jnp.where(jnp.isfinite(m), m, 0.0)
    ex = jnp.exp(s - m[seg])
    den = jax.ops.segment_sum(ex, seg, num)
    return ex / (den[seg] + 1e-16)


def _afp(x, ei, ea, batch, p, num_graphs):
    src, dst = ei[0], ei[1]
    n = x.shape[0]
    h = jax.nn.leaky_relu(x @ p['W_in'] + p['b_in'], 0.01)
    e = jax.nn.leaky_relu(ea @ p['W_edge'] + p['b_edge'], 0.01)
    for lp in p['layers']:
        z = jnp.concatenate([h[dst], h[src], e], axis=-1)
        sc = jax.nn.leaky_relu(z @ lp['a'], 0.01)
        al = _seg_softmax(sc, dst, n)
        msg = jax.ops.segment_sum(al * (h[src] @ lp['W_msg']), dst, n)
        h = _gru(jax.nn.elu(msg), h, lp['gru'])
    g = jax.ops.segment_sum(h, batch, num_graphs)
    mp = p['mol']
    for _ in range(N_MOL_LAYERS):
        z = jnp.concatenate([g[batch], h], axis=-1)
        sc = jax.nn.leaky_relu(z @ mp['a'], 0.01)
        al = _seg_softmax(sc, batch, num_graphs)
        msg = jax.ops.segment_sum(al * (h @ mp['W']), batch, num_graphs)
        g = _gru(jax.nn.elu(msg), g, mp['gru'])
    return g @ p['W_out'] + p['b_out']


def _bn(x):
    return (x - jnp.mean(x, 0)) / jnp.sqrt(jnp.var(x, 0) + 1e-5)


def _head_body(cc_ref, p1w, p1b, p2w, p2b, p3w, p3b, out_ref):
    cc = cc_ref[...]
    h1 = jax.nn.leaky_relu(cc @ p1w[...] + p1b[...], 0.001)
    mu = jnp.mean(h1, axis=0, keepdims=True)
    var = jnp.mean((h1 - mu) * (h1 - mu), axis=0, keepdims=True)
    h1 = (h1 - mu) / jnp.sqrt(var + 1e-5)
    h2 = jax.nn.leaky_relu(h1 @ p2w[...] + p2b[...], 0.001)
    out_ref[...] = h2 @ p3w[...] + p3b[...]


def _head(cc, P):
    mid = P['p1_W'].shape[1]
    out_dim = P['p3_W'].shape[1]
    return pl.pallas_call(
        _head_body,
        out_shape=jax.ShapeDtypeStruct((cc.shape[0], out_dim), jnp.float32),
    )(cc, P['p1_W'], P['p1_b'][None, :], P['p2_W'], P['p2_b'][None, :],
      P['p3_W'], P['p3_b'][None, :])


def kernel(data_x, data_edge_index, data_edge_attr, data_batch, frag_x,
           frag_edge_index, frag_edge_attr, frag_batch, junction_x,
           junction_edge_index, junction_edge_attr, junction_batch,
           motif_nodes, global_feats, params):
    P = params
    ex = jax.nn.leaky_relu(_bn(data_x @ P['o_node_W'] + P['o_node_b']), 0.01)
    ee = jax.nn.leaky_relu(_bn(data_edge_attr @ P['o_edge_W'] + P['o_edge_b']), 0.01)
    oo = jnp.concatenate([_afp(ex, data_edge_index, ee, data_batch, P['o_afp'][i], N_MOLS) for i in range(N_HEADS)], axis=-1)
    graph_origin = jax.nn.relu(_bn(oo @ P['o_att_W'] + P['o_att_b']))
    fo = jnp.concatenate([_afp(frag_x, frag_edge_index, frag_edge_attr, frag_batch, P['f_afp'][i], N_JCT) for i in range(N_HEADS)], axis=-1)
    graph_frag = jax.nn.relu(_bn(fo @ P['f_att_W'] + P['f_att_b']))
    me = jax.nn.leaky_relu(_bn(motif_nodes @ P['j_motif_W'] + P['j_motif_b']), 0.01)
    je = jax.nn.leaky_relu(_bn(junction_edge_attr @ P['j_edge_W'] + P['j_edge_b']), 0.01)
    xj = jnp.concatenate([junction_x, me], axis=-1)
    heads = [_afp(xj @ P['j_proj_W'][i] + P['j_proj_b'][i], junction_edge_index, je, junction_batch, P['j_afp'][i], N_MOLS) for i in range(N_HEADS)]
    super_new = jax.nn.relu(jnp.mean(jnp.stack(heads, 0), 0))
    frag_res = jax.ops.segment_sum(graph_frag, junction_batch, N_MOLS)
    cc = jnp.concatenate([graph_origin, frag_res, super_new, global_feats], axis=-1)
    return _head(cc, P)


# speedup vs baseline: 8.6878x; 7.9321x over previous
import functools

import jax
import jax.numpy as jnp
from jax import lax
from jax.experimental import pallas as pl
from jax.experimental.pallas import tpu as pltpu
from jax.experimental.pallas import tpu_sc as plsc

N_MOLS = 256
N_JCT = 2048
N_HEADS = 2
N_MOL_LAYERS = 2

C = 128
W = 80
NW = 32
NEG = -1e30


@functools.lru_cache(None)
def _attn_scatter(E_pad, R, Nd):
    Ew = E_pad // NW
    nchunks = Ew // C
    s8 = -(-(Nd // 16) // 8) * 8
    last = Nd - 15 * s8

    def _stripes(count):
        full, rem = divmod(count, C)
        out = [(k * C, C) for k in range(full)]
        if rem:
            out.append((full * C, rem))
        return out

    mesh = plsc.VectorSubcoreMesh(core_axis_name="c", subcore_axis_name="s")

    @functools.partial(
        pl.kernel,
        out_type=jax.ShapeDtypeStruct((2, Nd, W), jnp.float32),
        mesh=mesh,
        compiler_params=pltpu.CompilerParams(
            needs_layout_passes=False, use_tc_tiling_on_sc=False),
        scratch_types=[
            pltpu.VMEM_SHARED((Nd, W), jnp.float32),
            pltpu.VMEM((C,), jnp.int32),
            pltpu.VMEM((C,), jnp.int32),
            pltpu.VMEM((C,), jnp.float32),
            pltpu.VMEM((C,), jnp.float32),
            pltpu.VMEM((C,), jnp.float32),
            pltpu.VMEM((C,), jnp.float32),
            pltpu.VMEM((16,), jnp.float32),
            pltpu.VMEM((C, W), jnp.float32),
            pltpu.SemaphoreType.DMA,
        ],
    )
    def body(rows_hbm, src_hbm, dst_hbm, te_hbm, tdst_hbm, tsrc_hbm, m_hbm,
             out_hbm, acc, srcv, dstv, tev, exv, tdv, tsv, mv, rowbuf,
             sem):
        cid = lax.axis_index("c")
        sid = lax.axis_index("s")
        wid = sid * 2 + cid
        row0 = pl.multiple_of(sid * s8, 8)
        zero16 = jnp.zeros((16,), jnp.float32)

        def _striped(f):
            if last == s8:
                for off, sz in _stripes(s8):
                    f(off, sz)
            else:
                @pl.when(sid < 15)
                def _():
                    for off, sz in _stripes(s8):
                        f(off, sz)

                @pl.when(sid == 15)
                def _():
                    for off, sz in _stripes(last):
                        f(off, sz)

        def zrow(i, _):
            for q in range(W // 16):
                rowbuf[i, pl.ds(q * 16, 16)] = zero16
            return _
        lax.fori_loop(0, C, zrow, None)
        _striped(lambda off, sz: pltpu.sync_copy(
            rowbuf.at[pl.ds(0, sz)], acc.at[pl.ds(row0 + off, sz)]))

        pltpu.sync_copy(m_hbm, mv)
        plsc.subcore_barrier()

        def chunk(k, _):
            base = pl.multiple_of(wid * Ew + k * C, C)
            c1 = pltpu.async_copy(src_hbm.at[pl.ds(base, C)], srcv, sem)
            c2 = pltpu.async_copy(dst_hbm.at[pl.ds(base, C)], dstv, sem)
            c3 = pltpu.async_copy(te_hbm.at[pl.ds(base, C)], tev, sem)
            c1.wait(); c2.wait(); c3.wait()
            c4 = pltpu.async_copy(rows_hbm.at[srcv], rowbuf, sem)
            c5 = pltpu.async_copy(tdst_hbm.at[dstv], tdv, sem)
            c6 = pltpu.async_copy(tsrc_hbm.at[srcv], tsv, sem)
            c4.wait(); c5.wait(); c6.wait()
            m16 = mv[...]
            for j in range(C // 16):
                sc = (tdv[pl.ds(j * 16, 16)]
                      + tsv[pl.ds(j * 16, 16)]
                      + tev[pl.ds(j * 16, 16)])
                sc = jnp.where(sc >= 0.0, sc, sc * 0.01)
                exv[pl.ds(j * 16, 16)] = jnp.exp(sc - m16)
            for j in range(C // 16):
                ex16 = exv[pl.ds(j * 16, 16)]
                for i in range(16):
                    e = j * 16 + i
                    exs = ex16[i]
                    for q in range(W // 16):
                        rowbuf[e, pl.ds(q * 16, 16)] = (
                            rowbuf[e, pl.ds(q * 16, 16)] * exs)
            pltpu.sync_copy(rowbuf, acc.at[dstv], add=True)
            return _
        lax.fori_loop(0, nchunks, chunk, None)

        plsc.subcore_barrier()
        _striped(lambda off, sz: pltpu.sync_copy(
            acc.at[pl.ds(row0 + off, sz)],
            out_hbm.at[cid, pl.ds(row0 + off, sz)]))

    return body


def _seg_attn(rows64, src, dst, te, tdst, tsrc, m, Nd):
    R = rows64.shape[0]
    E = src.shape[0]
    E_pad = -(-E // (NW * C)) * (NW * C)
    pad = E_pad - E
    if pad:
        src = jnp.concatenate([src, jnp.zeros((pad,), jnp.int32)])
        dst = jnp.concatenate([dst, jnp.zeros((pad,), jnp.int32)])
        te = jnp.concatenate([te, jnp.full((pad,), NEG, jnp.float32)])
    rows = jnp.concatenate(
        [rows64, jnp.ones((R, 1), jnp.float32),
         jnp.zeros((R, W - 65), jnp.float32)], axis=1)
    m16 = jnp.full((16,), m, jnp.float32)
    out = _attn_scatter(E_pad, R, Nd)(
        rows, src.astype(jnp.int32), dst.astype(jnp.int32), te, tdst, tsrc,
        m16)
    return out[0] + out[1]


def _lk(x):
    return jax.nn.leaky_relu(x, 0.01)


def _shift(*ts):
    s = sum(jnp.max(t) for t in ts)
    return jnp.where(s >= 0.0, s, 0.01 * s)


def _gru(x, h, q):
    r = jax.nn.sigmoid(x @ q['Wr'] + h @ q['Ur'] + q['br'])
    z = jax.nn.sigmoid(x @ q['Wz'] + h @ q['Uz'] + q['bz'])
    n = jnp.tanh(x @ q['Wn'] + r * (h @ q['Un']) + q['bn'])
    return (1.0 - z) * n + z * h


def _finalize(out):
    return jax.nn.elu(out[:, :64] / (out[:, 64:65] + 1e-16))


def _afp(x, ei, ea, batch, p, num_graphs):
    src, dst = ei[0], ei[1]
    n = x.shape[0]
    h = _lk(x @ p['W_in'] + p['b_in'])
    e = _lk(ea @ p['W_edge'] + p['b_edge'])
    for lp in p['layers']:
        a1 = lp['a'][:64, 0]
        a2 = lp['a'][64:128, 0]
        a3 = lp['a'][128:, 0]
        tdst = h @ a1
        tsrc = h @ a2
        te = e @ a3
        m = _shift(tdst, tsrc, te)
        out = _seg_attn(h @ lp['W_msg'], src, dst, te, tdst, tsrc, m, n)
        h = _gru(_finalize(out), h, lp['gru'])
    iot = jnp.arange(n, dtype=jnp.int32)
    zn = jnp.zeros((n,), jnp.float32)
    g = _seg_attn(h, iot, batch, zn, jnp.zeros((num_graphs,), jnp.float32),
                  zn, 0.0, num_graphs)[:, :64]
    mp = p['mol']
    a_g = mp['a'][:64, 0]
    a_h = mp['a'][64:, 0]
    for _ in range(N_MOL_LAYERS):
        tg = g @ a_g
        th = h @ a_h
        m = _shift(tg, th)
        out = _seg_attn(h @ mp['W'], iot, batch, zn, tg, th, m, num_graphs)
        g = _gru(_finalize(out), g, mp['gru'])
    return g @ p['W_out'] + p['b_out']


def _bn(x):
    return (x - jnp.mean(x, 0)) / jnp.sqrt(jnp.var(x, 0) + 1e-5)


def _head_body(cc_ref, p1w, p1b, p2w, p2b, p3w, p3b, out_ref):
    cc = cc_ref[...]
    h1 = jax.nn.leaky_relu(cc @ p1w[...] + p1b[...], 0.001)
    mu = jnp.mean(h1, axis=0, keepdims=True)
    var = jnp.mean((h1 - mu) * (h1 - mu), axis=0, keepdims=True)
    h1 = (h1 - mu) / jnp.sqrt(var + 1e-5)
    h2 = jax.nn.leaky_relu(h1 @ p2w[...] + p2b[...], 0.001)
    out_ref[...] = h2 @ p3w[...] + p3b[...]


def _head(cc, P):
    out_dim = P['p3_W'].shape[1]
    return pl.pallas_call(
        _head_body,
        out_shape=jax.ShapeDtypeStruct((cc.shape[0], out_dim), jnp.float32),
    )(cc, P['p1_W'], P['p1_b'][None, :], P['p2_W'], P['p2_b'][None, :],
      P['p3_W'], P['p3_b'][None, :])


def kernel(data_x, data_edge_index, data_edge_attr, data_batch, frag_x,
           frag_edge_index, frag_edge_attr, frag_batch, junction_x,
           junction_edge_index, junction_edge_attr, junction_batch,
           motif_nodes, global_feats, params):
    P = params
    ex = _lk(_bn(data_x @ P['o_node_W'] + P['o_node_b']))
    ee = _lk(_bn(data_edge_attr @ P['o_edge_W'] + P['o_edge_b']))
    oo = jnp.concatenate(
        [_afp(ex, data_edge_index, ee, data_batch, P['o_afp'][i], N_MOLS)
         for i in range(N_HEADS)], axis=-1)
    graph_origin = jax.nn.relu(_bn(oo @ P['o_att_W'] + P['o_att_b']))
    fo = jnp.concatenate(
        [_afp(frag_x, frag_edge_index, frag_edge_attr, frag_batch,
              P['f_afp'][i], N_JCT) for i in range(N_HEADS)], axis=-1)
    graph_frag = jax.nn.relu(_bn(fo @ P['f_att_W'] + P['f_att_b']))
    me = _lk(_bn(motif_nodes @ P['j_motif_W'] + P['j_motif_b']))
    je = _lk(_bn(junction_edge_attr @ P['j_edge_W'] + P['j_edge_b']))
    xj = jnp.concatenate([junction_x, me], axis=-1)
    heads = [_afp(xj @ P['j_proj_W'][i] + P['j_proj_b'][i],
                  junction_edge_index, je, junction_batch, P['j_afp'][i],
                  N_MOLS) for i in range(N_HEADS)]
    super_new = jax.nn.relu(jnp.mean(jnp.stack(heads, 0), 0))
    jb = junction_batch.astype(jnp.int32)
    frag_res = _seg_attn(
        graph_frag, jnp.arange(N_JCT, dtype=jnp.int32), jb,
        jnp.zeros((N_JCT,), jnp.float32), jnp.zeros((N_MOLS,), jnp.float32),
        jnp.zeros((N_JCT,), jnp.float32), 0.0, N_MOLS)[:, :64]
    cc = jnp.concatenate([graph_origin, frag_res, super_new, global_feats],
                         axis=-1)
    return _head(cc, P)
